# baseline (speedup 1.0000x reference)
import functools
import math

import jax
import jax.numpy as jnp
from jax import lax
from jax.experimental import pallas as pl
from jax.experimental.pallas import tpu as pltpu
from jax.experimental.pallas import tpu_sc as plsc

_VOXEL_SIZES = [[0.08, 0.08, 0.1], [0.12, 0.12, 0.15], [0.2, 0.2, 0.25]]
_INTER_RADIUS = [0.15, 0.25, 0.4]
_INTRA_RADIUS = [0.2, 0.3, 0.5]
_CAPS = [2048, 1024, 512]
_PCAPS = [1792, 640, 128]
_PTOT = sum(_PCAPS)
_POFF = [0, 1792, 2432]
_N = 8192


def _grid_dims(vs):
    return (math.ceil(1.0 / vs[0]), math.ceil(1.0 / vs[1]), math.ceil(1.0 / vs[2]))


def _voxel_ids(p3, level):
    vs = _VOXEL_SIZES[level]
    nx, ny, nz = _grid_dims(vs)
    ii = jnp.clip(jnp.floor(p3[:, 0] / jnp.float32(vs[0])).astype(jnp.int32), 0, nx - 1)
    jj = jnp.clip(jnp.floor(p3[:, 1] / jnp.float32(vs[1])).astype(jnp.int32), 0, ny - 1)
    kk = jnp.clip(jnp.floor(p3[:, 2] / jnp.float32(vs[2])).astype(jnp.int32), 0, nz - 1)
    return (ii * ny + jj) * nz + kk


def _mask_mean(a, b, v, rr):
    a2 = jnp.sum(a * a, axis=1)[:, None]
    b2 = jnp.sum(b * b, axis=1)[None, :]
    ab = lax.dot_general(a, b, (((1,), (1,)), ((), ())),
                         preferred_element_type=jnp.float32)
    m = (a2 + b2 - 2.0 * ab <= jnp.float32(rr)).astype(jnp.float32)
    deg = jnp.sum(m, axis=1)
    agg = jnp.dot(m, v, preferred_element_type=jnp.float32)
    agg = agg / jnp.maximum(deg, 1.0)[:, None]
    return agg, deg


def _vox_tc(p4, level):
    cap = _CAPS[level]
    SB, PB = 256, 2048
    nsb, npb = cap // SB, _N // PB

    def body(p_ref, o_ref):
        i = pl.program_id(0)
        j = pl.program_id(1)
        p4b = p_ref[...]
        vid = _voxel_ids(p4b[:, 0:3], level)
        slots = SB * i + lax.broadcasted_iota(jnp.int32, (SB, PB), 0)
        oh = (slots == vid[None, :]).astype(jnp.float32)
        part = jnp.dot(oh, p4b, preferred_element_type=jnp.float32)

        @pl.when(j == 0)
        def _():
            o_ref[...] = part

        @pl.when(j > 0)
        def _():
            o_ref[...] = o_ref[...] + part

        @pl.when(j == npb - 1)
        def _():
            s = o_ref[...]
            cnt = s[:, 3:4]
            cent = s[:, 0:3] / jnp.maximum(cnt, 1.0)
            cent = jnp.where(cnt > 0, cent, jnp.float32(1e6))
            o_ref[...] = jnp.concatenate([cent, cnt], axis=1)

    out = pl.pallas_call(
        body,
        grid=(nsb, npb),
        in_specs=[pl.BlockSpec((PB, 4), lambda i, j: (j, 0))],
        out_specs=pl.BlockSpec((SB, 4), lambda i, j: (i, 0)),
        out_shape=jax.ShapeDtypeStruct((cap, 4), jnp.float32),
    )(p4)
    return out[:, 0:3]


def _dense_fused(sx, sy, sz, sn, p4, W, Wp, b2d):
    f32 = jnp.float32
    rr_i1 = float(_INTRA_RADIUS[0]) ** 2
    rr_i2 = float(_INTRA_RADIUS[1]) ** 2
    rr_i3 = float(_INTRA_RADIUS[2]) ** 2
    rr_e3 = float(_INTER_RADIUS[2]) ** 2
    rr_e2 = float(_INTER_RADIUS[1]) ** 2
    rr_e0 = float(_INTER_RADIUS[0]) ** 2
    BQ1, BQF = 256, 512
    NH = _PCAPS[0] // BQ1
    NF = _N // BQF

    def body(sx_r, sy_r, sz_r, sn_r, p_ref, w_ref, wp_ref, b_ref, o_ref, g1_o,
             c_s, h2_s, h1_s):
        t = pl.program_id(0)

        @pl.when(t == 0)
        def _():
            cnt = sn_r[...]
            d = jnp.maximum(cnt, 1.0)
            valid = cnt > 0.0
            big = jnp.float32(1e6)
            cxv = jnp.where(valid, sx_r[...] / d, big)
            cyv = jnp.where(valid, sy_r[...] / d, big)
            czv = jnp.where(valid, sz_r[...] / d, big)
            c_allv = jnp.concatenate([cxv, cyv, czv], axis=1)
            c_s[...] = c_allv
            c1 = c_allv[0:1792]
            c2 = c_allv[1792:2432]
            c3 = c_allv[2432:2560]
            h3, _ = _mask_mean(c3, c3, c3, rr_i3)
            s2, _ = _mask_mean(c2, c2, c2, rr_i2)
            a23, _ = _mask_mean(c2, c3, h3, rr_e3)
            h2_s[...] = s2 + a23

        @pl.when((t >= 1) & (t <= NH))
        def _():
            c_allv = c_s[...]
            c1 = c_allv[0:1792]
            c2 = c_allv[1792:2432]
            a = c_s[pl.ds((t - 1) * BQ1, BQ1), :]
            s1, _ = _mask_mean(a, c1, c1, rr_i1)
            a12, _ = _mask_mean(a, c2, h2_s[...], rr_e2)
            h1_s[pl.ds((t - 1) * BQ1, BQ1), :] = s1 + a12

        @pl.when(t == NH)
        def _():
            dn = (((1,), (1,)), ((), ()))
            g1_o[...] = lax.dot_general(h1_s[...], wp_ref[...][:, 6:9], dn,
                                        preferred_element_type=jnp.float32)

        @pl.when(t > NH)
        def _():
            pb = p_ref[...][:, 0:3]
            c1 = c_s[0:1792, :]
            h1v = h1_s[...]
            f2, deg0 = _mask_mean(pb, c1, h1v, rr_e0)
            w = w_ref[...]
            dn = (((1,), (1,)), ((), ()))
            out = lax.dot_general(pb, w[:, 0:3], dn, preferred_element_type=f32)
            out = out + lax.dot_general(f2, w[:, 3:6], dn, preferred_element_type=f32)
            out = out + (deg0 / 100.0)[:, None] * w[:, 9][None, :]
            o_ref[...] = out + b_ref[...]

    fmap = lambda t: (jnp.clip(t - (NH + 1), 0, NF - 1), 0)
    return pl.pallas_call(
        body,
        grid=(1 + NH + NF,),
        in_specs=[
            pl.BlockSpec((_PTOT, 1), lambda t: (0, 0)),
            pl.BlockSpec((_PTOT, 1), lambda t: (0, 0)),
            pl.BlockSpec((_PTOT, 1), lambda t: (0, 0)),
            pl.BlockSpec((_PTOT, 1), lambda t: (0, 0)),
            pl.BlockSpec((BQF, 4), fmap),
            pl.BlockSpec((100, 10), lambda t: (0, 0)),
            pl.BlockSpec((128, 10), lambda t: (0, 0)),
            pl.BlockSpec((1, 100), lambda t: (0, 0)),
        ],
        out_specs=[pl.BlockSpec((BQF, 100), fmap),
                   pl.BlockSpec((_PCAPS[0], 128), lambda t: (0, 0))],
        out_shape=[jax.ShapeDtypeStruct((_N, 100), jnp.float32),
                   jax.ShapeDtypeStruct((_PCAPS[0], 128), jnp.float32)],
        scratch_shapes=[
            pltpu.VMEM((_PTOT, 3), jnp.float32),
            pltpu.VMEM((640, 3), jnp.float32),
            pltpu.VMEM((1792, 3), jnp.float32),
        ],
    )(sx, sy, sz, sn, p4, W, Wp, b2d)


_NB = _N // 128


def _sc_voxelize(xs, ys, zs):
    mesh = plsc.VectorSubcoreMesh(core_axis_name="c", subcore_axis_name="s")
    grids = [_grid_dims(v) for v in _VOXEL_SIZES]
    f32, i32 = jnp.float32, jnp.int32

    @functools.partial(
        pl.kernel, mesh=mesh,
        out_type=[jax.ShapeDtypeStruct((_PTOT,), f32),
                  jax.ShapeDtypeStruct((_PTOT,), f32),
                  jax.ShapeDtypeStruct((_PTOT,), f32),
                  jax.ShapeDtypeStruct((_PTOT,), f32),
                  jax.ShapeDtypeStruct((_NB, 128), i32)],
        scratch_types=[
            pltpu.VMEM_SHARED((_NB, 128), i32),
            pltpu.VMEM_SHARED((_NB, 128), i32),
            pltpu.VMEM_SHARED((_NB, 128), i32),
            pltpu.VMEM_SHARED((1792,), f32), pltpu.VMEM_SHARED((1792,), f32),
            pltpu.VMEM_SHARED((1792,), f32), pltpu.VMEM_SHARED((1792,), f32),
            pltpu.VMEM_SHARED((640,), f32), pltpu.VMEM_SHARED((640,), f32),
            pltpu.VMEM_SHARED((640,), f32), pltpu.VMEM_SHARED((640,), f32),
            pltpu.VMEM_SHARED((128,), f32), pltpu.VMEM_SHARED((128,), f32),
            pltpu.VMEM_SHARED((128,), f32), pltpu.VMEM_SHARED((128,), f32),
            pltpu.VMEM((4, 128), f32),
            pltpu.VMEM((4, 128), f32),
            pltpu.VMEM((4, 128), f32),
            pltpu.VMEM((4, 128), i32),
            pltpu.VMEM((_NB, 128), i32),
            pltpu.VMEM((_NB, 128), f32),
            pltpu.VMEM((128,), f32),
            pltpu.VMEM((128,), f32),
        ],
    )
    def vox(xs_h, ys_h, zs_h,
            sx_o, sy_o, sz_o, sn_o, inv1_o,
            vshA, vshB, vshC,
            a1x, a1y, a1z, a1c, a2x, a2y, a2z, a2c, a3x, a3y, a3z, a3c,
            xv, yv, zv, vb, idx_v, val_v, ob, zb):
        cid = lax.axis_index("c")
        sid = lax.axis_index("s")
        rows = pl.ds(sid * 4, 4)
        pltpu.sync_copy(xs_h.at[rows], xv)
        pltpu.sync_copy(ys_h.at[rows], yv)
        pltpu.sync_copy(zs_h.at[rows], zv)
        for c16 in range(8):
            s = pl.ds(c16 * 16, 16)
            ob[s] = jnp.full((16,), 1.0, f32)
            zb[s] = jnp.full((16,), 0.0, f32)

        def compute_vids(level):
            vs = _VOXEL_SIZES[level]
            nx, ny, nz = grids[level]
            for r in range(4):
                for c16 in range(8):
                    s = pl.ds(c16 * 16, 16)
                    ii = jnp.clip((xv[r, s] / f32(vs[0])).astype(i32), 0, nx - 1)
                    jj = jnp.clip((yv[r, s] / f32(vs[1])).astype(i32), 0, ny - 1)
                    kk = jnp.clip((zv[r, s] / f32(vs[2])).astype(i32), 0, nz - 1)
                    vb[r, s] = (ii * ny + jj) * nz + kk

        @pl.when(cid == 0)
        def _():
            compute_vids(0)
            pltpu.sync_copy(vb, vshA.at[rows])
            pltpu.sync_copy(vb, inv1_o.at[rows])

        @pl.when(cid == 1)
        def _():
            compute_vids(1)
            pltpu.sync_copy(vb, vshB.at[rows])
            compute_vids(2)
            pltpu.sync_copy(vb, vshC.at[rows])

        roles = []
        for comp in range(4):
            src = [xs_h, ys_h, zs_h, None][comp]
            out = [sx_o, sy_o, sz_o, sn_o][comp]
            roles.append(((cid == 0) & (sid == comp), vshA, src,
                          [a1x, a1y, a1z, a1c][comp], 14, out, _POFF[0]))
            roles.append(((cid == 1) & (sid == comp), vshB, src,
                          [a2x, a2y, a2z, a2c][comp], 5, out, _POFF[1]))
            roles.append(((cid == 1) & (sid == comp + 4), vshC, src,
                          [a3x, a3y, a3z, a3c][comp], 1, out, _POFF[2]))

        for cond, _vsh, _src, acc, nblk, _out, _off in roles:
            @pl.when(cond)
            def _(acc=acc, nblk=nblk):
                def zstep(j, carry):
                    pltpu.sync_copy(zb, acc.at[pl.ds(j * 128, 128)])
                    return carry
                lax.fori_loop(0, nblk, zstep, 0)

        plsc.subcore_barrier()

        for cond, vsh, src, acc, nblk, out, off in roles:
            @pl.when(cond)
            def _(vsh=vsh, src=src, acc=acc, nblk=nblk, out=out, off=off):
                pltpu.sync_copy(vsh, idx_v)
                if src is not None:
                    pltpu.sync_copy(src, val_v)

                def sstep(j, carry):
                    if src is None:
                        pltpu.sync_copy(ob, acc.at[idx_v.at[j]], add=True)
                    else:
                        pltpu.sync_copy(val_v.at[j], acc.at[idx_v.at[j]], add=True)
                    return carry
                lax.fori_loop(0, _NB, sstep, 0)
                pltpu.sync_copy(acc, out.at[pl.ds(off, nblk * 128)])

    return vox(xs, ys, zs)


def _cent_tc(sx, sy, sz, sn):
    def body(sx_r, sy_r, sz_r, sn_r, cx_r, cy_r, cz_r):
        cnt = sn_r[...]
        d = jnp.maximum(cnt, 1.0)
        valid = cnt > 0.0
        big = jnp.float32(1e6)
        cx_r[...] = jnp.where(valid, sx_r[...] / d, big)
        cy_r[...] = jnp.where(valid, sy_r[...] / d, big)
        cz_r[...] = jnp.where(valid, sz_r[...] / d, big)

    sh = jax.ShapeDtypeStruct((28, 128), jnp.float32)
    return pl.pallas_call(body, out_shape=[sh, sh, sh])(sx, sy, sz, sn)


def _sc_gather(G1, inv_flat):
    mesh = plsc.VectorSubcoreMesh(core_axis_name="c", subcore_axis_name="s")

    @functools.partial(
        pl.kernel, mesh=mesh,
        out_type=jax.ShapeDtypeStruct((_N, 128), jnp.float32),
        scratch_types=[
            pltpu.VMEM((128,), jnp.int32),
            pltpu.VMEM((128, 128), jnp.float32),
            pltpu.SemaphoreType.DMA,
        ],
    )
    def g(g1_h, inv_h, out_h, idx_v, rows_v, sem):
        cid = lax.axis_index("c")
        sid = lax.axis_index("s")
        wid = sid * 2 + cid
        for half in range(2):
            base = wid * 256 + half * 128
            pltpu.sync_copy(inv_h.at[pl.ds(base, 128)], idx_v)
            pltpu.async_copy(g1_h.at[idx_v], rows_v, sem).wait()
            pltpu.sync_copy(rows_v, out_h.at[pl.ds(base, 128)])

    return g(G1, inv_flat)


def _add_tc(a, b):
    def body(a_ref, b_ref, o_ref):
        o_ref[...] = a_ref[...] + b_ref[...][:, 0:100]

    return pl.pallas_call(
        body,
        grid=(4,),
        in_specs=[pl.BlockSpec((2048, 100), lambda i: (i, 0)),
                  pl.BlockSpec((2048, 128), lambda i: (i, 0))],
        out_specs=pl.BlockSpec((2048, 100), lambda i: (i, 0)),
        out_shape=jax.ShapeDtypeStruct((_N, 100), jnp.float32),
    )(a, b)


def kernel(points, gt_bboxes_3d, gt_labels_3d, W, b):
    p = points[:, 0:3]
    p4 = jnp.concatenate([p, jnp.ones((_N, 1), jnp.float32)], axis=1)
    xs = jnp.reshape(p[:, 0], (_NB, 128))
    ys = jnp.reshape(p[:, 1], (_NB, 128))
    zs = jnp.reshape(p[:, 2], (_NB, 128))
    sx, sy, sz, sn, inv1 = _sc_voxelize(xs, ys, zs)
    Wp = jnp.concatenate([W, jnp.zeros((28, 10), jnp.float32)], axis=0)
    partial, G1 = _dense_fused(
        jnp.reshape(sx, (_PTOT, 1)), jnp.reshape(sy, (_PTOT, 1)),
        jnp.reshape(sz, (_PTOT, 1)), jnp.reshape(sn, (_PTOT, 1)),
        p4, W, Wp, jnp.reshape(b, (1, 100)))
    f3w = _sc_gather(G1, jnp.reshape(inv1, (_N,)))
    return _add_tc(partial, f3w)


# speedup vs baseline: 1.0195x; 1.0195x over previous
import functools
import math

import jax
import jax.numpy as jnp
from jax import lax
from jax.experimental import pallas as pl
from jax.experimental.pallas import tpu as pltpu
from jax.experimental.pallas import tpu_sc as plsc

_VOXEL_SIZES = [[0.08, 0.08, 0.1], [0.12, 0.12, 0.15], [0.2, 0.2, 0.25]]
_INTER_RADIUS = [0.15, 0.25, 0.4]
_INTRA_RADIUS = [0.2, 0.3, 0.5]
_CAPS = [2048, 1024, 512]
_PCAPS = [1792, 640, 128]
_PTOT = sum(_PCAPS)
_POFF = [0, 1792, 2432]
_N = 8192


def _grid_dims(vs):
    return (math.ceil(1.0 / vs[0]), math.ceil(1.0 / vs[1]), math.ceil(1.0 / vs[2]))


def _voxel_ids(p3, level):
    vs = _VOXEL_SIZES[level]
    nx, ny, nz = _grid_dims(vs)
    ii = jnp.clip(jnp.floor(p3[:, 0] / jnp.float32(vs[0])).astype(jnp.int32), 0, nx - 1)
    jj = jnp.clip(jnp.floor(p3[:, 1] / jnp.float32(vs[1])).astype(jnp.int32), 0, ny - 1)
    kk = jnp.clip(jnp.floor(p3[:, 2] / jnp.float32(vs[2])).astype(jnp.int32), 0, nz - 1)
    return (ii * ny + jj) * nz + kk


def _mask_mean(a, b, v, rr):
    a2 = jnp.sum(a * a, axis=1)[:, None]
    b2 = jnp.sum(b * b, axis=1)[None, :]
    ab = lax.dot_general(a, b, (((1,), (1,)), ((), ())),
                         preferred_element_type=jnp.float32)
    m = (a2 + b2 - 2.0 * ab <= jnp.float32(rr)).astype(jnp.float32)
    deg = jnp.sum(m, axis=1)
    agg = jnp.dot(m, v, preferred_element_type=jnp.float32)
    agg = agg / jnp.maximum(deg, 1.0)[:, None]
    return agg, deg


def _vox_tc(p4, level):
    cap = _CAPS[level]
    SB, PB = 256, 2048
    nsb, npb = cap // SB, _N // PB

    def body(p_ref, o_ref):
        i = pl.program_id(0)
        j = pl.program_id(1)
        p4b = p_ref[...]
        vid = _voxel_ids(p4b[:, 0:3], level)
        slots = SB * i + lax.broadcasted_iota(jnp.int32, (SB, PB), 0)
        oh = (slots == vid[None, :]).astype(jnp.float32)
        part = jnp.dot(oh, p4b, preferred_element_type=jnp.float32)

        @pl.when(j == 0)
        def _():
            o_ref[...] = part

        @pl.when(j > 0)
        def _():
            o_ref[...] = o_ref[...] + part

        @pl.when(j == npb - 1)
        def _():
            s = o_ref[...]
            cnt = s[:, 3:4]
            cent = s[:, 0:3] / jnp.maximum(cnt, 1.0)
            cent = jnp.where(cnt > 0, cent, jnp.float32(1e6))
            o_ref[...] = jnp.concatenate([cent, cnt], axis=1)

    out = pl.pallas_call(
        body,
        grid=(nsb, npb),
        in_specs=[pl.BlockSpec((PB, 4), lambda i, j: (j, 0))],
        out_specs=pl.BlockSpec((SB, 4), lambda i, j: (i, 0)),
        out_shape=jax.ShapeDtypeStruct((cap, 4), jnp.float32),
    )(p4)
    return out[:, 0:3]


def _dense_fused(sx, sy, sz, sn, p4, vid3d, W, b2d):
    f32 = jnp.float32
    rr_i1 = float(_INTRA_RADIUS[0]) ** 2
    rr_i2 = float(_INTRA_RADIUS[1]) ** 2
    rr_i3 = float(_INTRA_RADIUS[2]) ** 2
    rr_e3 = float(_INTER_RADIUS[2]) ** 2
    rr_e2 = float(_INTER_RADIUS[1]) ** 2
    rr_e0 = float(_INTER_RADIUS[0]) ** 2
    BQ1, BQF = 256, 512
    NH = _PCAPS[0] // BQ1
    NF = _N // BQF

    def body(sx_r, sy_r, sz_r, sn_r, p_ref, vid_ref, w_ref, b_ref, o_ref,
             c_s, h2_s, h1_s):
        t = pl.program_id(0)

        @pl.when(t == 0)
        def _():
            cnt = sn_r[...]
            d = jnp.maximum(cnt, 1.0)
            valid = cnt > 0.0
            big = jnp.float32(1e6)
            cxv = jnp.where(valid, sx_r[...] / d, big)
            cyv = jnp.where(valid, sy_r[...] / d, big)
            czv = jnp.where(valid, sz_r[...] / d, big)
            c_allv = jnp.concatenate([cxv, cyv, czv], axis=1)
            c_s[...] = c_allv
            c1 = c_allv[0:1792]
            c2 = c_allv[1792:2432]
            c3 = c_allv[2432:2560]
            h3, _ = _mask_mean(c3, c3, c3, rr_i3)
            s2, _ = _mask_mean(c2, c2, c2, rr_i2)
            a23, _ = _mask_mean(c2, c3, h3, rr_e3)
            h2_s[...] = s2 + a23

        @pl.when((t >= 1) & (t <= NH))
        def _():
            c_allv = c_s[...]
            c1 = c_allv[0:1792]
            c2 = c_allv[1792:2432]
            a = c_s[pl.ds((t - 1) * BQ1, BQ1), :]
            s1, _ = _mask_mean(a, c1, c1, rr_i1)
            a12, _ = _mask_mean(a, c2, h2_s[...], rr_e2)
            h1_s[pl.ds((t - 1) * BQ1, BQ1), :] = s1 + a12

        @pl.when(t > NH)
        def _():
            pb = p_ref[...][:, 0:3]
            c1 = c_s[0:1792, :]
            h1v = h1_s[...]
            f2, deg0 = _mask_mean(pb, c1, h1v, rr_e0)
            vid = vid_ref[0, 0, :]
            oh = (vid[:, None] == lax.broadcasted_iota(jnp.int32, (BQF, _PCAPS[0]), 1)
                  ).astype(f32)
            f3 = jnp.dot(oh, h1v, preferred_element_type=f32)
            w = w_ref[...]
            dn = (((1,), (1,)), ((), ()))
            out = lax.dot_general(pb, w[:, 0:3], dn, preferred_element_type=f32)
            out = out + lax.dot_general(f2, w[:, 3:6], dn, preferred_element_type=f32)
            out = out + lax.dot_general(f3, w[:, 6:9], dn, preferred_element_type=f32)
            out = out + (deg0 / 100.0)[:, None] * w[:, 9][None, :]
            o_ref[...] = out + b_ref[...]

    fmap = lambda t: (jnp.clip(t - (NH + 1), 0, NF - 1), 0)
    return pl.pallas_call(
        body,
        grid=(1 + NH + NF,),
        in_specs=[
            pl.BlockSpec((_PTOT, 1), lambda t: (0, 0)),
            pl.BlockSpec((_PTOT, 1), lambda t: (0, 0)),
            pl.BlockSpec((_PTOT, 1), lambda t: (0, 0)),
            pl.BlockSpec((_PTOT, 1), lambda t: (0, 0)),
            pl.BlockSpec((BQF, 4), fmap),
            pl.BlockSpec((1, 1, BQF), lambda t: (jnp.clip(t - (NH + 1), 0, NF - 1), 0, 0)),
            pl.BlockSpec((100, 10), lambda t: (0, 0)),
            pl.BlockSpec((1, 100), lambda t: (0, 0)),
        ],
        out_specs=pl.BlockSpec((BQF, 100), fmap),
        out_shape=jax.ShapeDtypeStruct((_N, 100), jnp.float32),
        scratch_shapes=[
            pltpu.VMEM((_PTOT, 3), jnp.float32),
            pltpu.VMEM((640, 3), jnp.float32),
            pltpu.VMEM((1792, 3), jnp.float32),
        ],
    )(sx, sy, sz, sn, p4, vid3d, W, b2d)


_NB = _N // 128


def _sc_voxelize(xs, ys, zs):
    mesh = plsc.VectorSubcoreMesh(core_axis_name="c", subcore_axis_name="s")
    grids = [_grid_dims(v) for v in _VOXEL_SIZES]
    f32, i32 = jnp.float32, jnp.int32

    @functools.partial(
        pl.kernel, mesh=mesh,
        out_type=[jax.ShapeDtypeStruct((_PTOT,), f32),
                  jax.ShapeDtypeStruct((_PTOT,), f32),
                  jax.ShapeDtypeStruct((_PTOT,), f32),
                  jax.ShapeDtypeStruct((_PTOT,), f32),
                  jax.ShapeDtypeStruct((_NB, 128), i32)],
        scratch_types=[
            pltpu.VMEM_SHARED((_NB, 128), i32),
            pltpu.VMEM_SHARED((_NB, 128), i32),
            pltpu.VMEM_SHARED((_NB, 128), i32),
            pltpu.VMEM_SHARED((1792,), f32), pltpu.VMEM_SHARED((1792,), f32),
            pltpu.VMEM_SHARED((1792,), f32), pltpu.VMEM_SHARED((1792,), f32),
            pltpu.VMEM_SHARED((640,), f32), pltpu.VMEM_SHARED((640,), f32),
            pltpu.VMEM_SHARED((640,), f32), pltpu.VMEM_SHARED((640,), f32),
            pltpu.VMEM_SHARED((128,), f32), pltpu.VMEM_SHARED((128,), f32),
            pltpu.VMEM_SHARED((128,), f32), pltpu.VMEM_SHARED((128,), f32),
            pltpu.VMEM((4, 128), f32),
            pltpu.VMEM((4, 128), f32),
            pltpu.VMEM((4, 128), f32),
            pltpu.VMEM((4, 128), i32),
            pltpu.VMEM((_NB, 128), i32),
            pltpu.VMEM((_NB, 128), f32),
            pltpu.VMEM((128,), f32),
            pltpu.VMEM((128,), f32),
        ],
    )
    def vox(xs_h, ys_h, zs_h,
            sx_o, sy_o, sz_o, sn_o, inv1_o,
            vshA, vshB, vshC,
            a1x, a1y, a1z, a1c, a2x, a2y, a2z, a2c, a3x, a3y, a3z, a3c,
            xv, yv, zv, vb, idx_v, val_v, ob, zb):
        cid = lax.axis_index("c")
        sid = lax.axis_index("s")
        rows = pl.ds(sid * 4, 4)
        pltpu.sync_copy(xs_h.at[rows], xv)
        pltpu.sync_copy(ys_h.at[rows], yv)
        pltpu.sync_copy(zs_h.at[rows], zv)
        for c16 in range(8):
            s = pl.ds(c16 * 16, 16)
            ob[s] = jnp.full((16,), 1.0, f32)
            zb[s] = jnp.full((16,), 0.0, f32)

        def compute_vids(level):
            vs = _VOXEL_SIZES[level]
            nx, ny, nz = grids[level]
            for r in range(4):
                for c16 in range(8):
                    s = pl.ds(c16 * 16, 16)
                    ii = jnp.clip((xv[r, s] / f32(vs[0])).astype(i32), 0, nx - 1)
                    jj = jnp.clip((yv[r, s] / f32(vs[1])).astype(i32), 0, ny - 1)
                    kk = jnp.clip((zv[r, s] / f32(vs[2])).astype(i32), 0, nz - 1)
                    vb[r, s] = (ii * ny + jj) * nz + kk

        @pl.when(cid == 0)
        def _():
            compute_vids(0)
            pltpu.sync_copy(vb, vshA.at[rows])
            pltpu.sync_copy(vb, inv1_o.at[rows])

        @pl.when(cid == 1)
        def _():
            compute_vids(1)
            pltpu.sync_copy(vb, vshB.at[rows])
            compute_vids(2)
            pltpu.sync_copy(vb, vshC.at[rows])

        roles = []
        for comp in range(4):
            src = [xs_h, ys_h, zs_h, None][comp]
            out = [sx_o, sy_o, sz_o, sn_o][comp]
            roles.append(((cid == 0) & (sid == comp), vshA, src,
                          [a1x, a1y, a1z, a1c][comp], 14, out, _POFF[0]))
            roles.append(((cid == 1) & (sid == comp), vshB, src,
                          [a2x, a2y, a2z, a2c][comp], 5, out, _POFF[1]))
            roles.append(((cid == 1) & (sid == comp + 4), vshC, src,
                          [a3x, a3y, a3z, a3c][comp], 1, out, _POFF[2]))

        for cond, _vsh, _src, acc, nblk, _out, _off in roles:
            @pl.when(cond)
            def _(acc=acc, nblk=nblk):
                def zstep(j, carry):
                    pltpu.sync_copy(zb, acc.at[pl.ds(j * 128, 128)])
                    return carry
                lax.fori_loop(0, nblk, zstep, 0)

        plsc.subcore_barrier()

        for cond, vsh, src, acc, nblk, out, off in roles:
            @pl.when(cond)
            def _(vsh=vsh, src=src, acc=acc, nblk=nblk, out=out, off=off):
                pltpu.sync_copy(vsh, idx_v)
                if src is not None:
                    pltpu.sync_copy(src, val_v)

                def sstep(j, carry):
                    if src is None:
                        pltpu.sync_copy(ob, acc.at[idx_v.at[j]], add=True)
                    else:
                        pltpu.sync_copy(val_v.at[j], acc.at[idx_v.at[j]], add=True)
                    return carry
                lax.fori_loop(0, _NB, sstep, 0)
                pltpu.sync_copy(acc, out.at[pl.ds(off, nblk * 128)])

    return vox(xs, ys, zs)


def _cent_tc(sx, sy, sz, sn):
    def body(sx_r, sy_r, sz_r, sn_r, cx_r, cy_r, cz_r):
        cnt = sn_r[...]
        d = jnp.maximum(cnt, 1.0)
        valid = cnt > 0.0
        big = jnp.float32(1e6)
        cx_r[...] = jnp.where(valid, sx_r[...] / d, big)
        cy_r[...] = jnp.where(valid, sy_r[...] / d, big)
        cz_r[...] = jnp.where(valid, sz_r[...] / d, big)

    sh = jax.ShapeDtypeStruct((28, 128), jnp.float32)
    return pl.pallas_call(body, out_shape=[sh, sh, sh])(sx, sy, sz, sn)


def kernel(points, gt_bboxes_3d, gt_labels_3d, W, b):
    p = points[:, 0:3]
    p4 = jnp.concatenate([p, jnp.ones((_N, 1), jnp.float32)], axis=1)
    xs = jnp.reshape(p[:, 0], (_NB, 128))
    ys = jnp.reshape(p[:, 1], (_NB, 128))
    zs = jnp.reshape(p[:, 2], (_NB, 128))
    sx, sy, sz, sn, inv1 = _sc_voxelize(xs, ys, zs)
    vid3d = jnp.reshape(inv1, (_N // 512, 1, 512))
    return _dense_fused(jnp.reshape(sx, (_PTOT, 1)), jnp.reshape(sy, (_PTOT, 1)),
                        jnp.reshape(sz, (_PTOT, 1)), jnp.reshape(sn, (_PTOT, 1)),
                        p4, vid3d, W, jnp.reshape(b, (1, 100)))


# speedup vs baseline: 1.0207x; 1.0012x over previous
import functools
import math

import jax
import jax.numpy as jnp
from jax import lax
from jax.experimental import pallas as pl
from jax.experimental.pallas import tpu as pltpu
from jax.experimental.pallas import tpu_sc as plsc

_VOXEL_SIZES = [[0.08, 0.08, 0.1], [0.12, 0.12, 0.15], [0.2, 0.2, 0.25]]
_INTER_RADIUS = [0.15, 0.25, 0.4]
_INTRA_RADIUS = [0.2, 0.3, 0.5]
_CAPS = [2048, 1024, 512]
_PCAPS = [1792, 640, 128]
_PTOT = sum(_PCAPS)
_POFF = [0, 1792, 2432]
_N = 8192


def _grid_dims(vs):
    return (math.ceil(1.0 / vs[0]), math.ceil(1.0 / vs[1]), math.ceil(1.0 / vs[2]))


def _mask_mean(a, b, v, rr):
    a2 = jnp.sum(a * a, axis=1)[:, None]
    b2 = jnp.sum(b * b, axis=1)[None, :]
    ab = lax.dot_general(a, b, (((1,), (1,)), ((), ())),
                         preferred_element_type=jnp.float32)
    m = (a2 + b2 - 2.0 * ab <= jnp.float32(rr)).astype(jnp.float32)
    deg = jnp.sum(m, axis=1)
    agg = jnp.dot(m, v, preferred_element_type=jnp.float32)
    agg = agg / jnp.maximum(deg, 1.0)[:, None]
    return agg, deg


def _dense_fused(sx, sy, sz, sn, p4, vid3d, W, b2d):
    f32 = jnp.float32
    rr_i1 = float(_INTRA_RADIUS[0]) ** 2
    rr_i2 = float(_INTRA_RADIUS[1]) ** 2
    rr_i3 = float(_INTRA_RADIUS[2]) ** 2
    rr_e3 = float(_INTER_RADIUS[2]) ** 2
    rr_e2 = float(_INTER_RADIUS[1]) ** 2
    rr_e0 = float(_INTER_RADIUS[0]) ** 2
    BQ1, BQF = 256, 512
    NH = _PCAPS[0] // BQ1
    NF = _N // BQF

    def body(sx_r, sy_r, sz_r, sn_r, p_ref, vid_ref, w_ref, b_ref, o_ref,
             c_s, h2_s, h1_s):
        t = pl.program_id(0)

        @pl.when(t == 0)
        def _():
            cnt = sn_r[...]
            d = jnp.maximum(cnt, 1.0)
            valid = cnt > 0.0
            big = jnp.float32(1e6)
            cxv = jnp.where(valid, sx_r[...] / d, big)
            cyv = jnp.where(valid, sy_r[...] / d, big)
            czv = jnp.where(valid, sz_r[...] / d, big)
            c_allv = jnp.concatenate([cxv, cyv, czv], axis=1)
            c_s[...] = c_allv
            c1 = c_allv[0:1792]
            c2 = c_allv[1792:2432]
            c3 = c_allv[2432:2560]
            h3, _ = _mask_mean(c3, c3, c3, rr_i3)
            s2, _ = _mask_mean(c2, c2, c2, rr_i2)
            a23, _ = _mask_mean(c2, c3, h3, rr_e3)
            h2_s[...] = s2 + a23

        @pl.when((t >= 1) & (t <= NH))
        def _():
            c_allv = c_s[...]
            c1 = c_allv[0:1792]
            c2 = c_allv[1792:2432]
            a = c_s[pl.ds((t - 1) * BQ1, BQ1), :]
            s1, _ = _mask_mean(a, c1, c1, rr_i1)
            a12, _ = _mask_mean(a, c2, h2_s[...], rr_e2)
            h1_s[pl.ds((t - 1) * BQ1, BQ1), :] = s1 + a12

        @pl.when(t > NH)
        def _():
            pb = p_ref[...][:, 0:3]
            c1 = c_s[0:1792, :]
            h1v = h1_s[...]
            f2, deg0 = _mask_mean(pb, c1, h1v, rr_e0)
            vid = vid_ref[0, 0, :]
            oh = (vid[:, None] == lax.broadcasted_iota(jnp.int32, (BQF, _PCAPS[0]), 1)
                  ).astype(f32)
            f3 = jnp.dot(oh, h1v, preferred_element_type=f32)
            w = w_ref[...]
            dn = (((1,), (1,)), ((), ()))
            out = lax.dot_general(pb, w[:, 0:3], dn, preferred_element_type=f32)
            out = out + lax.dot_general(f2, w[:, 3:6], dn, preferred_element_type=f32)
            out = out + lax.dot_general(f3, w[:, 6:9], dn, preferred_element_type=f32)
            out = out + (deg0 / 100.0)[:, None] * w[:, 9][None, :]
            o_ref[...] = out + b_ref[...]

    fmap = lambda t: (jnp.clip(t - (NH + 1), 0, NF - 1), 0)
    return pl.pallas_call(
        body,
        grid=(1 + NH + NF,),
        in_specs=[
            pl.BlockSpec((_PTOT, 1), lambda t: (0, 0)),
            pl.BlockSpec((_PTOT, 1), lambda t: (0, 0)),
            pl.BlockSpec((_PTOT, 1), lambda t: (0, 0)),
            pl.BlockSpec((_PTOT, 1), lambda t: (0, 0)),
            pl.BlockSpec((BQF, 4), fmap),
            pl.BlockSpec((1, 1, BQF), lambda t: (jnp.clip(t - (NH + 1), 0, NF - 1), 0, 0)),
            pl.BlockSpec((100, 10), lambda t: (0, 0)),
            pl.BlockSpec((1, 100), lambda t: (0, 0)),
        ],
        out_specs=pl.BlockSpec((BQF, 100), fmap),
        out_shape=jax.ShapeDtypeStruct((_N, 100), jnp.float32),
        scratch_shapes=[
            pltpu.VMEM((_PTOT, 3), jnp.float32),
            pltpu.VMEM((640, 3), jnp.float32),
            pltpu.VMEM((1792, 3), jnp.float32),
        ],
    )(sx, sy, sz, sn, p4, vid3d, W, b2d)


_NB = _N // 128


def _sc_voxelize(xs, ys, zs):
    mesh = plsc.VectorSubcoreMesh(core_axis_name="c", subcore_axis_name="s")
    grids = [_grid_dims(v) for v in _VOXEL_SIZES]
    f32, i32 = jnp.float32, jnp.int32

    @functools.partial(
        pl.kernel, mesh=mesh,
        out_type=[jax.ShapeDtypeStruct((_PTOT,), f32),
                  jax.ShapeDtypeStruct((_PTOT,), f32),
                  jax.ShapeDtypeStruct((_PTOT,), f32),
                  jax.ShapeDtypeStruct((_PTOT,), f32),
                  jax.ShapeDtypeStruct((_NB, 128), i32)],
        scratch_types=[
            pltpu.VMEM_SHARED((_NB, 128), i32),
            pltpu.VMEM_SHARED((_NB, 128), i32),
            pltpu.VMEM_SHARED((_NB, 128), i32),
            pltpu.VMEM_SHARED((1792,), f32), pltpu.VMEM_SHARED((1792,), f32),
            pltpu.VMEM_SHARED((1792,), f32), pltpu.VMEM_SHARED((1792,), f32),
            pltpu.VMEM_SHARED((640,), f32), pltpu.VMEM_SHARED((640,), f32),
            pltpu.VMEM_SHARED((640,), f32), pltpu.VMEM_SHARED((640,), f32),
            pltpu.VMEM_SHARED((128,), f32), pltpu.VMEM_SHARED((128,), f32),
            pltpu.VMEM_SHARED((128,), f32), pltpu.VMEM_SHARED((128,), f32),
            pltpu.VMEM((4, 128), f32),
            pltpu.VMEM((4, 128), f32),
            pltpu.VMEM((4, 128), f32),
            pltpu.VMEM((4, 128), i32),
            pltpu.VMEM((_NB, 128), i32),
            pltpu.VMEM((_NB, 128), f32),
            pltpu.VMEM((128,), f32),
            pltpu.VMEM((128,), f32),
        ],
    )
    def vox(xs_h, ys_h, zs_h,
            sx_o, sy_o, sz_o, sn_o, inv1_o,
            vshA, vshB, vshC,
            a1x, a1y, a1z, a1c, a2x, a2y, a2z, a2c, a3x, a3y, a3z, a3c,
            xv, yv, zv, vb, idx_v, val_v, ob, zb):
        cid = lax.axis_index("c")
        sid = lax.axis_index("s")
        rows = pl.ds(sid * 4, 4)
        pltpu.sync_copy(xs_h.at[rows], xv)
        pltpu.sync_copy(ys_h.at[rows], yv)
        pltpu.sync_copy(zs_h.at[rows], zv)
        for c16 in range(8):
            s = pl.ds(c16 * 16, 16)
            ob[s] = jnp.full((16,), 1.0, f32)
            zb[s] = jnp.full((16,), 0.0, f32)

        def compute_vids(level):
            vs = _VOXEL_SIZES[level]
            nx, ny, nz = grids[level]
            for r in range(4):
                for c16 in range(8):
                    s = pl.ds(c16 * 16, 16)
                    ii = jnp.clip((xv[r, s] / f32(vs[0])).astype(i32), 0, nx - 1)
                    jj = jnp.clip((yv[r, s] / f32(vs[1])).astype(i32), 0, ny - 1)
                    kk = jnp.clip((zv[r, s] / f32(vs[2])).astype(i32), 0, nz - 1)
                    vb[r, s] = (ii * ny + jj) * nz + kk

        @pl.when(cid == 0)
        def _():
            compute_vids(0)
            pltpu.sync_copy(vb, vshA.at[rows])
            pltpu.sync_copy(vb, inv1_o.at[rows])

        @pl.when(cid == 1)
        def _():
            compute_vids(1)
            pltpu.sync_copy(vb, vshB.at[rows])
            compute_vids(2)
            pltpu.sync_copy(vb, vshC.at[rows])

        roles = []
        for comp in range(4):
            src = [xs_h, ys_h, zs_h, None][comp]
            out = [sx_o, sy_o, sz_o, sn_o][comp]
            roles.append(((cid == 0) & (sid == comp), vshA, src,
                          [a1x, a1y, a1z, a1c][comp], 14, out, _POFF[0]))
            roles.append(((cid == 1) & (sid == comp), vshB, src,
                          [a2x, a2y, a2z, a2c][comp], 5, out, _POFF[1]))
            roles.append(((cid == 1) & (sid == comp + 4), vshC, src,
                          [a3x, a3y, a3z, a3c][comp], 1, out, _POFF[2]))

        for cond, _vsh, _src, acc, nblk, _out, _off in roles:
            @pl.when(cond)
            def _(acc=acc, nblk=nblk):
                def zstep(j, carry):
                    pltpu.sync_copy(zb, acc.at[pl.ds(j * 128, 128)])
                    return carry
                lax.fori_loop(0, nblk, zstep, 0)

        plsc.subcore_barrier()

        for cond, vsh, src, acc, nblk, out, off in roles:
            @pl.when(cond)
            def _(vsh=vsh, src=src, acc=acc, nblk=nblk, out=out, off=off):
                pltpu.sync_copy(vsh, idx_v)
                if src is not None:
                    pltpu.sync_copy(src, val_v)

                def sstep(j, carry):
                    if src is None:
                        pltpu.sync_copy(ob, acc.at[idx_v.at[j]], add=True)
                    else:
                        pltpu.sync_copy(val_v.at[j], acc.at[idx_v.at[j]], add=True)
                    return carry
                lax.fori_loop(0, _NB, sstep, 0)
                pltpu.sync_copy(acc, out.at[pl.ds(off, nblk * 128)])

    return vox(xs, ys, zs)


def _cent_tc(sx, sy, sz, sn):
    def body(sx_r, sy_r, sz_r, sn_r, cx_r, cy_r, cz_r):
        cnt = sn_r[...]
        d = jnp.maximum(cnt, 1.0)
        valid = cnt > 0.0
        big = jnp.float32(1e6)
        cx_r[...] = jnp.where(valid, sx_r[...] / d, big)
        cy_r[...] = jnp.where(valid, sy_r[...] / d, big)
        cz_r[...] = jnp.where(valid, sz_r[...] / d, big)

    sh = jax.ShapeDtypeStruct((28, 128), jnp.float32)
    return pl.pallas_call(body, out_shape=[sh, sh, sh])(sx, sy, sz, sn)


def kernel(points, gt_bboxes_3d, gt_labels_3d, W, b):
    p = points[:, 0:3]
    p4 = jnp.concatenate([p, jnp.ones((_N, 1), jnp.float32)], axis=1)
    xs = jnp.reshape(p[:, 0], (_NB, 128))
    ys = jnp.reshape(p[:, 1], (_NB, 128))
    zs = jnp.reshape(p[:, 2], (_NB, 128))
    sx, sy, sz, sn, inv1 = _sc_voxelize(xs, ys, zs)
    vid3d = jnp.reshape(inv1, (_N // 512, 1, 512))
    return _dense_fused(jnp.reshape(sx, (_PTOT, 1)), jnp.reshape(sy, (_PTOT, 1)),
                        jnp.reshape(sz, (_PTOT, 1)), jnp.reshape(sn, (_PTOT, 1)),
                        p4, vid3d, W, jnp.reshape(b, (1, 100)))
